# Initial kernel scaffold; baseline (speedup 1.0000x reference)
import jax
import jax.numpy as jnp
from jax.experimental import pallas as pl


def kernel(x, Wg1, bg1, Wg2, bg2, We1, be1, We2, be2):
    raise NotImplementedError("write your pallas kernel here")


# speedup vs baseline: 1.0386x; 1.0386x over previous
import functools

import jax
import jax.numpy as jnp
from jax.experimental import pallas as pl

NUM_EXPERTS = 8
TOP_K = 2
D_MODEL = 768
D_GATE_HID = 2 * D_MODEL
D_FF = 4 * D_MODEL
LB_COEF = 0.01

T_TILE = 256
FF_TILE = 512
N_FF = D_FF // FF_TILE


def _gate_kernel(x_ref, wg1_ref, bg1_ref, wg2_ref, bg2_ref,
                 cw_ref, usage_ref):
    x = x_ref[...]
    h = jnp.maximum(jnp.dot(x, wg1_ref[...],
                            preferred_element_type=jnp.float32)
                    + bg1_ref[...], 0.0)
    logits = jnp.dot(h, wg2_ref[...],
                     preferred_element_type=jnp.float32) + bg2_ref[...]
    m = jnp.max(logits, axis=-1, keepdims=True)
    e = jnp.exp(logits - m)
    scores = e / jnp.sum(e, axis=-1, keepdims=True)

    lane = jax.lax.broadcasted_iota(jnp.int32, scores.shape, 1)
    big = jnp.int32(NUM_EXPERTS)
    m1 = jnp.max(scores, axis=-1, keepdims=True)
    i1 = jnp.min(jnp.where(scores == m1, lane, big), axis=-1, keepdims=True)
    oh1 = (lane == i1).astype(scores.dtype)
    masked = jnp.where(lane == i1, -jnp.inf, scores)
    m2 = jnp.max(masked, axis=-1, keepdims=True)
    i2 = jnp.min(jnp.where(masked == m2, lane, big), axis=-1, keepdims=True)
    oh2 = (lane == i2).astype(scores.dtype)
    cw_ref[...] = (oh1 * m1 + oh2 * m2) / (m1 + m2)

    @pl.when(pl.program_id(0) == 0)
    def _init():
        usage_ref[...] = jnp.zeros_like(usage_ref)

    usage_ref[...] += jnp.sum(scores, axis=0, keepdims=True)


def _expert_kernel(x_ref, we1_ref, be1_ref, we2_ref, be2_ref, cw_ref,
                   out_ref):
    e = pl.program_id(1)
    f = pl.program_id(2)

    cw = cw_ref[...]
    lane = jax.lax.broadcasted_iota(jnp.int32, cw.shape, 1)
    w = jnp.sum(jnp.where(lane == e, cw, 0.0), axis=1, keepdims=True)

    @pl.when(jnp.logical_and(e == 0, f == 0))
    def _init():
        out_ref[...] = jnp.zeros_like(out_ref)

    @pl.when(f == 0)
    def _bias():
        out_ref[...] += w * be2_ref[0]

    h = jnp.dot(x_ref[...], we1_ref[0],
                preferred_element_type=jnp.float32) + be1_ref[0]
    h = 0.5 * h * (1.0 + jax.lax.erf(h * 0.7071067811865476))
    out_ref[...] += w * jnp.dot(h, we2_ref[0],
                                preferred_element_type=jnp.float32)


@jax.jit
def kernel(x, Wg1, bg1, Wg2, bg2, We1, be1, We2, be2):
    B, S, D = x.shape
    T = B * S
    x_flat = x.reshape(T, D)
    n_t = T // T_TILE

    combine_w, usage_sum = pl.pallas_call(
        _gate_kernel,
        grid=(n_t,),
        in_specs=[
            pl.BlockSpec((T_TILE, D_MODEL), lambda t: (t, 0)),
            pl.BlockSpec((D_MODEL, D_GATE_HID), lambda t: (0, 0)),
            pl.BlockSpec((1, D_GATE_HID), lambda t: (0, 0)),
            pl.BlockSpec((D_GATE_HID, NUM_EXPERTS), lambda t: (0, 0)),
            pl.BlockSpec((1, NUM_EXPERTS), lambda t: (0, 0)),
        ],
        out_specs=(
            pl.BlockSpec((T_TILE, NUM_EXPERTS), lambda t: (t, 0)),
            pl.BlockSpec((1, NUM_EXPERTS), lambda t: (0, 0)),
        ),
        out_shape=(
            jax.ShapeDtypeStruct((T, NUM_EXPERTS), jnp.float32),
            jax.ShapeDtypeStruct((1, NUM_EXPERTS), jnp.float32),
        ),
    )(x_flat, Wg1, bg1.reshape(1, -1), Wg2, bg2.reshape(1, -1))

    out = pl.pallas_call(
        _expert_kernel,
        grid=(n_t, NUM_EXPERTS, N_FF),
        in_specs=[
            pl.BlockSpec((T_TILE, D_MODEL), lambda t, e, f: (t, 0)),
            pl.BlockSpec((1, D_MODEL, FF_TILE), lambda t, e, f: (e, 0, f)),
            pl.BlockSpec((1, 1, FF_TILE), lambda t, e, f: (e * N_FF + f, 0, 0)),
            pl.BlockSpec((1, FF_TILE, D_MODEL), lambda t, e, f: (e, f, 0)),
            pl.BlockSpec((1, 1, D_MODEL), lambda t, e, f: (e, 0, 0)),
            pl.BlockSpec((T_TILE, NUM_EXPERTS), lambda t, e, f: (t, 0)),
        ],
        out_specs=pl.BlockSpec((T_TILE, D_MODEL), lambda t, e, f: (t, 0)),
        out_shape=jax.ShapeDtypeStruct((T, D_MODEL), jnp.float32),
    )(x_flat, We1, be1.reshape(NUM_EXPERTS * N_FF, 1, FF_TILE),
      We2, be2.reshape(NUM_EXPERTS, 1, D_MODEL), combine_w)

    usage = usage_sum[0] / T
    ideal = 1.0 / NUM_EXPERTS
    lb_loss = LB_COEF * jnp.mean((usage - ideal) ** 2)
    return out.reshape(B, S, D), lb_loss


# speedup vs baseline: 2.5061x; 2.4131x over previous
import functools

import jax
import jax.numpy as jnp
from jax import lax
from jax.experimental import pallas as pl
from jax.experimental.pallas import tpu as pltpu
from jax.experimental.pallas import tpu_sc as plsc

NUM_EXPERTS = 8
TOP_K = 2
D_MODEL = 768
D_GATE_HID = 2 * D_MODEL
D_FF = 4 * D_MODEL
LB_COEF = 0.01

T_TILE = 256
R_TILE = 256
FF_TILE = 512
N_FF = D_FF // FF_TILE

NC, NS = 2, 16
NW = NC * NS


def _gate_kernel(x_ref, wg1_ref, bg1_ref, wg2_ref, bg2_ref,
                 i1_ref, i2_ref, w1_ref, w2_ref, usage_ref):
    x = x_ref[...]
    h = jnp.maximum(jnp.dot(x, wg1_ref[...],
                            preferred_element_type=jnp.float32)
                    + bg1_ref[...], 0.0)
    logits = jnp.dot(h, wg2_ref[...],
                     preferred_element_type=jnp.float32) + bg2_ref[...]
    m = jnp.max(logits, axis=-1, keepdims=True)
    e = jnp.exp(logits - m)
    scores = e / jnp.sum(e, axis=-1, keepdims=True)

    lane = jax.lax.broadcasted_iota(jnp.int32, scores.shape, 1)
    big = jnp.int32(NUM_EXPERTS)
    m1 = jnp.max(scores, axis=-1, keepdims=True)
    i1 = jnp.min(jnp.where(scores == m1, lane, big), axis=-1, keepdims=True)
    masked = jnp.where(lane == i1, -jnp.inf, scores)
    m2 = jnp.max(masked, axis=-1, keepdims=True)
    i2 = jnp.min(jnp.where(masked == m2, lane, big), axis=-1, keepdims=True)

    i1_ref[...] = i1
    i2_ref[...] = i2
    s = m1 + m2
    w1_ref[...] = m1 / s
    w2_ref[...] = m2 / s

    @pl.when(pl.program_id(0) == 0)
    def _init():
        usage_ref[...] = jnp.zeros_like(usage_ref)

    usage_ref[...] += jnp.sum(scores, axis=0, keepdims=True)


def _gmm_kernel(e_map_ref, x_ref, we1_ref, be1_ref, we2_ref, be2_ref,
                out_ref):
    f = pl.program_id(1)

    @pl.when(f == 0)
    def _init():
        out_ref[...] = jnp.broadcast_to(be2_ref[0], out_ref.shape)

    h = jnp.dot(x_ref[...], we1_ref[0],
                preferred_element_type=jnp.float32) + be1_ref[0]
    h = 0.5 * h * (1.0 + jax.lax.erf(h * 0.7071067811865476))
    out_ref[...] += jnp.dot(h, we2_ref[0],
                            preferred_element_type=jnp.float32)


def _make_dispatch(n_tok, p_rows):
    tpw = n_tok // NW
    mesh = plsc.VectorSubcoreMesh(core_axis_name="c", subcore_axis_name="s",
                                  num_cores=NC, num_subcores=NS)

    @functools.partial(
        pl.kernel,
        out_type=jax.ShapeDtypeStruct((p_rows, D_MODEL), jnp.float32),
        mesh=mesh,
        scratch_types=[
            pltpu.VMEM((tpw,), jnp.int32),
            pltpu.VMEM((tpw,), jnp.int32),
            pltpu.VMEM((tpw, D_MODEL), jnp.float32),
            pltpu.SemaphoreType.DMA,
        ],
    )
    def dispatch(x_hbm, idx0_hbm, idx1_hbm, out_hbm,
                 idx0_v, idx1_v, rows_v, sem):
        wid = lax.axis_index("s") * NC + lax.axis_index("c")
        base = wid * tpw
        pltpu.sync_copy(idx0_hbm.at[pl.ds(base, tpw)], idx0_v)
        pltpu.sync_copy(idx1_hbm.at[pl.ds(base, tpw)], idx1_v)
        pltpu.sync_copy(x_hbm.at[pl.ds(base, tpw)], rows_v)
        pltpu.async_copy(rows_v, out_hbm.at[idx0_v], sem).wait()
        pltpu.async_copy(rows_v, out_hbm.at[idx1_v], sem).wait()

    return dispatch


def _make_combine(n_tok):
    sub = 64
    n_sub = n_tok // (NW * sub)
    mesh = plsc.VectorSubcoreMesh(core_axis_name="c", subcore_axis_name="s",
                                  num_cores=NC, num_subcores=NS)

    @functools.partial(
        pl.kernel,
        out_type=jax.ShapeDtypeStruct((n_tok, D_MODEL), jnp.float32),
        mesh=mesh,
        scratch_types=[
            pltpu.VMEM((sub,), jnp.int32),
            pltpu.VMEM((sub,), jnp.int32),
            pltpu.VMEM((sub, 16), jnp.float32),
            pltpu.VMEM((sub, 16), jnp.float32),
            pltpu.VMEM((sub, D_MODEL), jnp.float32),
            pltpu.VMEM((sub, D_MODEL), jnp.float32),
            pltpu.SemaphoreType.DMA,
        ],
    )
    def combine(y_hbm, r0_hbm, r1_hbm, w0_hbm, w1_hbm, out_hbm,
                r0_v, r1_v, w0_v, w1_v, a_v, b_v, sem):
        wid = lax.axis_index("s") * NC + lax.axis_index("c")
        for s in range(n_sub):
            base = (wid * n_sub + s) * sub
            pltpu.sync_copy(r0_hbm.at[pl.ds(base, sub)], r0_v)
            pltpu.sync_copy(r1_hbm.at[pl.ds(base, sub)], r1_v)
            pltpu.sync_copy(w0_hbm.at[pl.ds(base, sub)], w0_v)
            pltpu.sync_copy(w1_hbm.at[pl.ds(base, sub)], w1_v)
            pltpu.async_copy(y_hbm.at[r0_v], a_v, sem).wait()
            pltpu.async_copy(y_hbm.at[r1_v], b_v, sem).wait()

            def body(i, _):
                wa = w0_v[i, :]
                wb = w1_v[i, :]
                for c in range(D_MODEL // 16):
                    a_v[i, pl.ds(c * 16, 16)] = (
                        wa * a_v[i, pl.ds(c * 16, 16)]
                        + wb * b_v[i, pl.ds(c * 16, 16)])
                return 0

            lax.fori_loop(0, sub, body, 0)
            pltpu.sync_copy(a_v, out_hbm.at[pl.ds(base, sub)])

    return combine


@jax.jit
def kernel(x, Wg1, bg1, Wg2, bg2, We1, be1, We2, be2):
    B, S, D = x.shape
    T = B * S
    x_flat = x.reshape(T, D)
    n_t = T // T_TILE

    i1, i2, w1, w2, usage_sum = pl.pallas_call(
        _gate_kernel,
        grid=(n_t,),
        in_specs=[
            pl.BlockSpec((T_TILE, D_MODEL), lambda t: (t, 0)),
            pl.BlockSpec((D_MODEL, D_GATE_HID), lambda t: (0, 0)),
            pl.BlockSpec((1, D_GATE_HID), lambda t: (0, 0)),
            pl.BlockSpec((D_GATE_HID, NUM_EXPERTS), lambda t: (0, 0)),
            pl.BlockSpec((1, NUM_EXPERTS), lambda t: (0, 0)),
        ],
        out_specs=(
            pl.BlockSpec((T_TILE, 1), lambda t: (t, 0)),
            pl.BlockSpec((T_TILE, 1), lambda t: (t, 0)),
            pl.BlockSpec((T_TILE, 1), lambda t: (t, 0)),
            pl.BlockSpec((T_TILE, 1), lambda t: (t, 0)),
            pl.BlockSpec((1, NUM_EXPERTS), lambda t: (0, 0)),
        ),
        out_shape=(
            jax.ShapeDtypeStruct((T, 1), jnp.int32),
            jax.ShapeDtypeStruct((T, 1), jnp.int32),
            jax.ShapeDtypeStruct((T, 1), jnp.float32),
            jax.ShapeDtypeStruct((T, 1), jnp.float32),
            jax.ShapeDtypeStruct((1, NUM_EXPERTS), jnp.float32),
        ),
    )(x_flat, Wg1, bg1.reshape(1, -1), Wg2, bg2.reshape(1, -1))

    e_flat = jnp.concatenate([i1, i2], axis=1).reshape(-1)
    oh = (e_flat[:, None] == jnp.arange(NUM_EXPERTS)[None, :]).astype(jnp.int32)
    cum = jnp.cumsum(oh, axis=0)
    rank = jnp.take_along_axis(cum, e_flat[:, None], axis=1)[:, 0] - 1
    counts = cum[-1]
    psize = ((counts + R_TILE - 1) // R_TILE) * R_TILE
    pstart = jnp.concatenate([jnp.zeros((1,), jnp.int32),
                              jnp.cumsum(psize)[:-1].astype(jnp.int32)])
    dest = pstart[e_flat] + rank
    dest2 = dest.reshape(T, TOP_K)
    idx0 = dest2[:, 0]
    idx1 = dest2[:, 1]

    P = T * TOP_K + NUM_EXPERTS * R_TILE
    n_tiles = P // R_TILE
    t_starts = jnp.arange(n_tiles, dtype=jnp.int32) * R_TILE
    e_of_t = (jnp.sum((pstart[None, :] <= t_starts[:, None]), axis=1) - 1
              ).astype(jnp.int32)

    x_sorted = _make_dispatch(T, P)(x_flat, idx0, idx1)

    y_sorted = pl.pallas_call(
        _gmm_kernel,
        grid_spec=pltpu.PrefetchScalarGridSpec(
            num_scalar_prefetch=1,
            grid=(n_tiles, N_FF),
            in_specs=[
                pl.BlockSpec((R_TILE, D_MODEL), lambda t, f, em: (t, 0)),
                pl.BlockSpec((1, D_MODEL, FF_TILE),
                             lambda t, f, em: (em[t], 0, f)),
                pl.BlockSpec((1, 1, FF_TILE),
                             lambda t, f, em: (em[t] * N_FF + f, 0, 0)),
                pl.BlockSpec((1, FF_TILE, D_MODEL),
                             lambda t, f, em: (em[t], f, 0)),
                pl.BlockSpec((1, 1, D_MODEL),
                             lambda t, f, em: (em[t], 0, 0)),
            ],
            out_specs=pl.BlockSpec((R_TILE, D_MODEL), lambda t, f, em: (t, 0)),
        ),
        out_shape=jax.ShapeDtypeStruct((P, D_MODEL), jnp.float32),
    )(e_of_t, x_sorted, We1,
      be1.reshape(NUM_EXPERTS * N_FF, 1, FF_TILE),
      We2, be2.reshape(NUM_EXPERTS, 1, D_MODEL))

    w0b = jnp.broadcast_to(w1, (T, 16))
    w1b = jnp.broadcast_to(w2, (T, 16))
    out = _make_combine(T)(y_sorted, idx0, idx1, w0b, w1b)

    usage = usage_sum[0] / T
    ideal = 1.0 / NUM_EXPERTS
    lb_loss = LB_COEF * jnp.mean((usage - ideal) ** 2)
    return out.reshape(B, S, D), lb_loss


# speedup vs baseline: 3.5418x; 1.4133x over previous
import functools

import jax
import jax.numpy as jnp
from jax import lax
from jax.experimental import pallas as pl
from jax.experimental.pallas import tpu as pltpu
from jax.experimental.pallas import tpu_sc as plsc

NUM_EXPERTS = 8
TOP_K = 2
D_MODEL = 768
D_GATE_HID = 2 * D_MODEL
D_FF = 4 * D_MODEL
LB_COEF = 0.01

T_TILE = 256
R_TILE = 256
FF_TILE = 512
N_FF = D_FF // FF_TILE

NC, NS = 2, 16
NW = NC * NS


def _gate_kernel(x_ref, wg1_ref, bg1_ref, wg2_ref, bg2_ref,
                 i1_ref, i2_ref, w1_ref, w2_ref, usage_ref):
    x = x_ref[...]
    h = jnp.maximum(jnp.dot(x, wg1_ref[...],
                            preferred_element_type=jnp.float32)
                    + bg1_ref[...], 0.0)
    logits = jnp.dot(h, wg2_ref[...],
                     preferred_element_type=jnp.float32) + bg2_ref[...]
    m = jnp.max(logits, axis=-1, keepdims=True)
    e = jnp.exp(logits - m)
    scores = e / jnp.sum(e, axis=-1, keepdims=True)

    lane = jax.lax.broadcasted_iota(jnp.int32, scores.shape, 1)
    big = jnp.int32(NUM_EXPERTS)
    m1 = jnp.max(scores, axis=-1, keepdims=True)
    i1 = jnp.min(jnp.where(scores == m1, lane, big), axis=-1, keepdims=True)
    masked = jnp.where(lane == i1, -jnp.inf, scores)
    m2 = jnp.max(masked, axis=-1, keepdims=True)
    i2 = jnp.min(jnp.where(masked == m2, lane, big), axis=-1, keepdims=True)

    i1_ref[...] = i1
    i2_ref[...] = i2
    s = m1 + m2
    w1_ref[...] = m1 / s
    w2_ref[...] = m2 / s

    @pl.when(pl.program_id(0) == 0)
    def _init():
        usage_ref[...] = jnp.zeros_like(usage_ref)

    usage_ref[...] += jnp.sum(scores, axis=0, keepdims=True)


def _gmm_kernel(e_map_ref, x_ref, we1_ref, be1_ref, we2_ref, be2_ref,
                out_ref):
    x_bf = x_ref[...].astype(jnp.bfloat16)
    h = jnp.dot(x_bf, we1_ref[0],
                preferred_element_type=jnp.float32) + be1_ref[0]
    h = 0.5 * h * (1.0 + jax.lax.erf(h * 0.7071067811865476))
    out_ref[...] = jnp.dot(h.astype(jnp.bfloat16), we2_ref[0],
                           preferred_element_type=jnp.float32) + be2_ref[0]


def _make_dispatch(n_tok, p_rows):
    tpw = n_tok // NW
    mesh = plsc.VectorSubcoreMesh(core_axis_name="c", subcore_axis_name="s",
                                  num_cores=NC, num_subcores=NS)

    @functools.partial(
        pl.kernel,
        out_type=jax.ShapeDtypeStruct((p_rows, D_MODEL), jnp.float32),
        mesh=mesh,
        scratch_types=[
            pltpu.VMEM((tpw,), jnp.int32),
            pltpu.VMEM((tpw,), jnp.int32),
            pltpu.VMEM((tpw, D_MODEL), jnp.float32),
            pltpu.SemaphoreType.DMA,
        ],
    )
    def dispatch(x_hbm, idx0_hbm, idx1_hbm, out_hbm,
                 idx0_v, idx1_v, rows_v, sem):
        wid = lax.axis_index("s") * NC + lax.axis_index("c")
        base = wid * tpw
        pltpu.sync_copy(idx0_hbm.at[pl.ds(base, tpw)], idx0_v)
        pltpu.sync_copy(idx1_hbm.at[pl.ds(base, tpw)], idx1_v)
        pltpu.sync_copy(x_hbm.at[pl.ds(base, tpw)], rows_v)
        pltpu.async_copy(rows_v, out_hbm.at[idx0_v], sem).wait()
        pltpu.async_copy(rows_v, out_hbm.at[idx1_v], sem).wait()

    return dispatch


def _make_combine(n_tok):
    sub = 64
    n_sub = n_tok // (NW * sub)
    mesh = plsc.VectorSubcoreMesh(core_axis_name="c", subcore_axis_name="s",
                                  num_cores=NC, num_subcores=NS)

    @functools.partial(
        pl.kernel,
        out_type=jax.ShapeDtypeStruct((n_tok, D_MODEL), jnp.float32),
        mesh=mesh,
        scratch_types=[
            pltpu.VMEM((sub,), jnp.int32),
            pltpu.VMEM((sub,), jnp.int32),
            pltpu.VMEM((sub, 16), jnp.float32),
            pltpu.VMEM((sub, 16), jnp.float32),
            pltpu.VMEM((sub, D_MODEL), jnp.float32),
            pltpu.VMEM((sub, D_MODEL), jnp.float32),
            pltpu.SemaphoreType.DMA,
        ],
    )
    def combine(y_hbm, r0_hbm, r1_hbm, w0_hbm, w1_hbm, out_hbm,
                r0_v, r1_v, w0_v, w1_v, a_v, b_v, sem):
        wid = lax.axis_index("s") * NC + lax.axis_index("c")
        for s in range(n_sub):
            base = (wid * n_sub + s) * sub
            pltpu.sync_copy(r0_hbm.at[pl.ds(base, sub)], r0_v)
            pltpu.sync_copy(r1_hbm.at[pl.ds(base, sub)], r1_v)
            pltpu.sync_copy(w0_hbm.at[pl.ds(base, sub)], w0_v)
            pltpu.sync_copy(w1_hbm.at[pl.ds(base, sub)], w1_v)
            pltpu.async_copy(y_hbm.at[r0_v], a_v, sem).wait()
            pltpu.async_copy(y_hbm.at[r1_v], b_v, sem).wait()

            def body(i, _):
                wa = w0_v[i, :]
                wb = w1_v[i, :]
                for c in range(D_MODEL // 16):
                    a_v[i, pl.ds(c * 16, 16)] = (
                        wa * a_v[i, pl.ds(c * 16, 16)]
                        + wb * b_v[i, pl.ds(c * 16, 16)])
                return 0

            lax.fori_loop(0, sub, body, 0)
            pltpu.sync_copy(a_v, out_hbm.at[pl.ds(base, sub)])

    return combine


@jax.jit
def kernel(x, Wg1, bg1, Wg2, bg2, We1, be1, We2, be2):
    B, S, D = x.shape
    T = B * S
    x_flat = x.reshape(T, D)
    n_t = T // T_TILE

    i1, i2, w1, w2, usage_sum = pl.pallas_call(
        _gate_kernel,
        grid=(n_t,),
        in_specs=[
            pl.BlockSpec((T_TILE, D_MODEL), lambda t: (t, 0)),
            pl.BlockSpec((D_MODEL, D_GATE_HID), lambda t: (0, 0)),
            pl.BlockSpec((1, D_GATE_HID), lambda t: (0, 0)),
            pl.BlockSpec((D_GATE_HID, NUM_EXPERTS), lambda t: (0, 0)),
            pl.BlockSpec((1, NUM_EXPERTS), lambda t: (0, 0)),
        ],
        out_specs=(
            pl.BlockSpec((T_TILE, 1), lambda t: (t, 0)),
            pl.BlockSpec((T_TILE, 1), lambda t: (t, 0)),
            pl.BlockSpec((T_TILE, 1), lambda t: (t, 0)),
            pl.BlockSpec((T_TILE, 1), lambda t: (t, 0)),
            pl.BlockSpec((1, NUM_EXPERTS), lambda t: (0, 0)),
        ),
        out_shape=(
            jax.ShapeDtypeStruct((T, 1), jnp.int32),
            jax.ShapeDtypeStruct((T, 1), jnp.int32),
            jax.ShapeDtypeStruct((T, 1), jnp.float32),
            jax.ShapeDtypeStruct((T, 1), jnp.float32),
            jax.ShapeDtypeStruct((1, NUM_EXPERTS), jnp.float32),
        ),
    )(x_flat, Wg1, bg1.reshape(1, -1), Wg2, bg2.reshape(1, -1))

    e_flat = jnp.concatenate([i1, i2], axis=1).reshape(-1)
    oh = (e_flat[:, None] == jnp.arange(NUM_EXPERTS)[None, :]).astype(jnp.int32)
    cum = jnp.cumsum(oh, axis=0)
    rank = jnp.take_along_axis(cum, e_flat[:, None], axis=1)[:, 0] - 1
    counts = cum[-1]
    psize = ((counts + R_TILE - 1) // R_TILE) * R_TILE
    pstart = jnp.concatenate([jnp.zeros((1,), jnp.int32),
                              jnp.cumsum(psize)[:-1].astype(jnp.int32)])
    dest = pstart[e_flat] + rank
    dest2 = dest.reshape(T, TOP_K)
    idx0 = dest2[:, 0]
    idx1 = dest2[:, 1]

    P = T * TOP_K + NUM_EXPERTS * R_TILE
    n_tiles = P // R_TILE
    t_starts = jnp.arange(n_tiles, dtype=jnp.int32) * R_TILE
    e_of_t = (jnp.sum((pstart[None, :] <= t_starts[:, None]), axis=1) - 1
              ).astype(jnp.int32)

    x_sorted = _make_dispatch(T, P)(x_flat, idx0, idx1)

    y_sorted = pl.pallas_call(
        _gmm_kernel,
        grid_spec=pltpu.PrefetchScalarGridSpec(
            num_scalar_prefetch=1,
            grid=(n_tiles,),
            in_specs=[
                pl.BlockSpec((R_TILE, D_MODEL), lambda t, em: (t, 0)),
                pl.BlockSpec((1, D_MODEL, D_FF), lambda t, em: (em[t], 0, 0)),
                pl.BlockSpec((1, 1, D_FF), lambda t, em: (em[t], 0, 0)),
                pl.BlockSpec((1, D_FF, D_MODEL), lambda t, em: (em[t], 0, 0)),
                pl.BlockSpec((1, 1, D_MODEL), lambda t, em: (em[t], 0, 0)),
            ],
            out_specs=pl.BlockSpec((R_TILE, D_MODEL), lambda t, em: (t, 0)),
        ),
        out_shape=jax.ShapeDtypeStruct((P, D_MODEL), jnp.float32),
    )(e_of_t, x_sorted, We1.astype(jnp.bfloat16),
      be1.reshape(NUM_EXPERTS, 1, D_FF),
      We2.astype(jnp.bfloat16), be2.reshape(NUM_EXPERTS, 1, D_MODEL))

    w0b = jnp.broadcast_to(w1, (T, 16))
    w1b = jnp.broadcast_to(w2, (T, 16))
    out = _make_combine(T)(y_sorted, idx0, idx1, w0b, w1b)

    usage = usage_sum[0] / T
    ideal = 1.0 / NUM_EXPERTS
    lb_loss = LB_COEF * jnp.mean((usage - ideal) ** 2)
    return out.reshape(B, S, D), lb_loss


# speedup vs baseline: 3.5676x; 1.0073x over previous
import functools

import jax
import jax.numpy as jnp
from jax import lax
from jax.experimental import pallas as pl
from jax.experimental.pallas import tpu as pltpu
from jax.experimental.pallas import tpu_sc as plsc

NUM_EXPERTS = 8
TOP_K = 2
D_MODEL = 768
D_GATE_HID = 2 * D_MODEL
D_FF = 4 * D_MODEL
LB_COEF = 0.01

T_TILE = 256
R_TILE = 256
FF_TILE = 512
N_FF = D_FF // FF_TILE

NC, NS = 2, 16
NW = NC * NS


def _gate_kernel(x_ref, wg1_ref, bg1_ref, wg2_ref, bg2_ref,
                 i1_ref, i2_ref, w1_ref, w2_ref, usage_ref):
    x = x_ref[...].astype(jnp.bfloat16)
    h = jnp.maximum(jnp.dot(x, wg1_ref[...],
                            preferred_element_type=jnp.float32)
                    + bg1_ref[...], 0.0)
    logits = jnp.dot(h.astype(jnp.bfloat16), wg2_ref[...],
                     preferred_element_type=jnp.float32) + bg2_ref[...]
    m = jnp.max(logits, axis=-1, keepdims=True)
    e = jnp.exp(logits - m)
    scores = e / jnp.sum(e, axis=-1, keepdims=True)

    lane = jax.lax.broadcasted_iota(jnp.int32, scores.shape, 1)
    big = jnp.int32(NUM_EXPERTS)
    m1 = jnp.max(scores, axis=-1, keepdims=True)
    i1 = jnp.min(jnp.where(scores == m1, lane, big), axis=-1, keepdims=True)
    masked = jnp.where(lane == i1, -jnp.inf, scores)
    m2 = jnp.max(masked, axis=-1, keepdims=True)
    i2 = jnp.min(jnp.where(masked == m2, lane, big), axis=-1, keepdims=True)

    i1_ref[...] = i1
    i2_ref[...] = i2
    s = m1 + m2
    w1_ref[...] = m1 / s
    w2_ref[...] = m2 / s

    @pl.when(pl.program_id(0) == 0)
    def _init():
        usage_ref[...] = jnp.zeros_like(usage_ref)

    usage_ref[...] += jnp.sum(scores, axis=0, keepdims=True)


def _gmm_kernel(e_map_ref, x_ref, we1_ref, be1_ref, we2_ref, be2_ref,
                out_ref):
    x_bf = x_ref[...].astype(jnp.bfloat16)
    h = jnp.dot(x_bf, we1_ref[0],
                preferred_element_type=jnp.float32) + be1_ref[0]
    h = 0.5 * h * (1.0 + jax.lax.erf(h * 0.7071067811865476))
    out_ref[...] = jnp.dot(h.astype(jnp.bfloat16), we2_ref[0],
                           preferred_element_type=jnp.float32) + be2_ref[0]


def _make_dispatch(n_tok, p_rows):
    tpw = n_tok // NW
    mesh = plsc.VectorSubcoreMesh(core_axis_name="c", subcore_axis_name="s",
                                  num_cores=NC, num_subcores=NS)

    @functools.partial(
        pl.kernel,
        out_type=jax.ShapeDtypeStruct((p_rows, D_MODEL), jnp.float32),
        mesh=mesh,
        scratch_types=[
            pltpu.VMEM((tpw,), jnp.int32),
            pltpu.VMEM((tpw,), jnp.int32),
            pltpu.VMEM((tpw, D_MODEL), jnp.float32),
            pltpu.SemaphoreType.DMA,
        ],
    )
    def dispatch(x_hbm, idx0_hbm, idx1_hbm, out_hbm,
                 idx0_v, idx1_v, rows_v, sem):
        wid = lax.axis_index("s") * NC + lax.axis_index("c")
        base = wid * tpw
        c0 = pltpu.async_copy(idx0_hbm.at[pl.ds(base, tpw)], idx0_v, sem)
        c1 = pltpu.async_copy(idx1_hbm.at[pl.ds(base, tpw)], idx1_v, sem)
        c2 = pltpu.async_copy(x_hbm.at[pl.ds(base, tpw)], rows_v, sem)
        c0.wait()
        c1.wait()
        c2.wait()
        s0 = pltpu.async_copy(rows_v, out_hbm.at[idx0_v], sem)
        s1 = pltpu.async_copy(rows_v, out_hbm.at[idx1_v], sem)
        s0.wait()
        s1.wait()

    return dispatch


def _make_combine(n_tok):
    sub = 64
    n_sub = n_tok // (NW * sub)
    mesh = plsc.VectorSubcoreMesh(core_axis_name="c", subcore_axis_name="s",
                                  num_cores=NC, num_subcores=NS)

    @functools.partial(
        pl.kernel,
        out_type=jax.ShapeDtypeStruct((n_tok, D_MODEL), jnp.float32),
        mesh=mesh,
        scratch_types=[
            pltpu.VMEM((sub,), jnp.int32),
            pltpu.VMEM((sub,), jnp.int32),
            pltpu.VMEM((sub, 16), jnp.float32),
            pltpu.VMEM((sub, 16), jnp.float32),
            pltpu.VMEM((sub, D_MODEL), jnp.float32),
            pltpu.VMEM((sub, D_MODEL), jnp.float32),
            pltpu.SemaphoreType.DMA,
        ],
    )
    def combine(y_hbm, r0_hbm, r1_hbm, w0_hbm, w1_hbm, out_hbm,
                r0_v, r1_v, w0_v, w1_v, a_v, b_v, sem):
        wid = lax.axis_index("s") * NC + lax.axis_index("c")
        for s in range(n_sub):
            base = (wid * n_sub + s) * sub
            c0 = pltpu.async_copy(r0_hbm.at[pl.ds(base, sub)], r0_v, sem)
            c1 = pltpu.async_copy(r1_hbm.at[pl.ds(base, sub)], r1_v, sem)
            c2 = pltpu.async_copy(w0_hbm.at[pl.ds(base, sub)], w0_v, sem)
            c3 = pltpu.async_copy(w1_hbm.at[pl.ds(base, sub)], w1_v, sem)
            c0.wait()
            c1.wait()
            c2.wait()
            c3.wait()
            g0 = pltpu.async_copy(y_hbm.at[r0_v], a_v, sem)
            g1 = pltpu.async_copy(y_hbm.at[r1_v], b_v, sem)
            g0.wait()
            g1.wait()

            def body(i, _):
                wa = w0_v[i, :]
                wb = w1_v[i, :]
                for c in range(D_MODEL // 16):
                    a_v[i, pl.ds(c * 16, 16)] = (
                        wa * a_v[i, pl.ds(c * 16, 16)]
                        + wb * b_v[i, pl.ds(c * 16, 16)])
                return 0

            lax.fori_loop(0, sub, body, 0)
            pltpu.sync_copy(a_v, out_hbm.at[pl.ds(base, sub)])

    return combine


@jax.jit
def kernel(x, Wg1, bg1, Wg2, bg2, We1, be1, We2, be2):
    B, S, D = x.shape
    T = B * S
    x_flat = x.reshape(T, D)
    n_t = T // T_TILE

    i1, i2, w1, w2, usage_sum = pl.pallas_call(
        _gate_kernel,
        grid=(n_t,),
        in_specs=[
            pl.BlockSpec((T_TILE, D_MODEL), lambda t: (t, 0)),
            pl.BlockSpec((D_MODEL, D_GATE_HID), lambda t: (0, 0)),
            pl.BlockSpec((1, D_GATE_HID), lambda t: (0, 0)),
            pl.BlockSpec((D_GATE_HID, NUM_EXPERTS), lambda t: (0, 0)),
            pl.BlockSpec((1, NUM_EXPERTS), lambda t: (0, 0)),
        ],
        out_specs=(
            pl.BlockSpec((T_TILE, 1), lambda t: (t, 0)),
            pl.BlockSpec((T_TILE, 1), lambda t: (t, 0)),
            pl.BlockSpec((T_TILE, 1), lambda t: (t, 0)),
            pl.BlockSpec((T_TILE, 1), lambda t: (t, 0)),
            pl.BlockSpec((1, NUM_EXPERTS), lambda t: (0, 0)),
        ),
        out_shape=(
            jax.ShapeDtypeStruct((T, 1), jnp.int32),
            jax.ShapeDtypeStruct((T, 1), jnp.int32),
            jax.ShapeDtypeStruct((T, 1), jnp.float32),
            jax.ShapeDtypeStruct((T, 1), jnp.float32),
            jax.ShapeDtypeStruct((1, NUM_EXPERTS), jnp.float32),
        ),
    )(x_flat, Wg1.astype(jnp.bfloat16), bg1.reshape(1, -1),
      Wg2.astype(jnp.bfloat16), bg2.reshape(1, -1))

    e_flat = jnp.concatenate([i1, i2], axis=1).reshape(-1)
    oh = (e_flat[:, None] == jnp.arange(NUM_EXPERTS)[None, :]).astype(jnp.int32)
    cum = jnp.cumsum(oh, axis=0)
    rank = jnp.take_along_axis(cum, e_flat[:, None], axis=1)[:, 0] - 1
    counts = cum[-1]
    psize = ((counts + R_TILE - 1) // R_TILE) * R_TILE
    pstart = jnp.concatenate([jnp.zeros((1,), jnp.int32),
                              jnp.cumsum(psize)[:-1].astype(jnp.int32)])
    dest = pstart[e_flat] + rank
    dest2 = dest.reshape(T, TOP_K)
    idx0 = dest2[:, 0]
    idx1 = dest2[:, 1]

    P = T * TOP_K + NUM_EXPERTS * R_TILE
    n_tiles = P // R_TILE
    t_starts = jnp.arange(n_tiles, dtype=jnp.int32) * R_TILE
    e_of_t = (jnp.sum((pstart[None, :] <= t_starts[:, None]), axis=1) - 1
              ).astype(jnp.int32)

    x_sorted = _make_dispatch(T, P)(x_flat, idx0, idx1)

    y_sorted = pl.pallas_call(
        _gmm_kernel,
        grid_spec=pltpu.PrefetchScalarGridSpec(
            num_scalar_prefetch=1,
            grid=(n_tiles,),
            in_specs=[
                pl.BlockSpec((R_TILE, D_MODEL), lambda t, em: (t, 0)),
                pl.BlockSpec((1, D_MODEL, D_FF), lambda t, em: (em[t], 0, 0)),
                pl.BlockSpec((1, 1, D_FF), lambda t, em: (em[t], 0, 0)),
                pl.BlockSpec((1, D_FF, D_MODEL), lambda t, em: (em[t], 0, 0)),
                pl.BlockSpec((1, 1, D_MODEL), lambda t, em: (em[t], 0, 0)),
            ],
            out_specs=pl.BlockSpec((R_TILE, D_MODEL), lambda t, em: (t, 0)),
        ),
        out_shape=jax.ShapeDtypeStruct((P, D_MODEL), jnp.float32),
    )(e_of_t, x_sorted, We1.astype(jnp.bfloat16),
      be1.reshape(NUM_EXPERTS, 1, D_FF),
      We2.astype(jnp.bfloat16), be2.reshape(NUM_EXPERTS, 1, D_MODEL))

    w0b = jnp.broadcast_to(w1, (T, 16))
    w1b = jnp.broadcast_to(w2, (T, 16))
    out = _make_combine(T)(y_sorted, idx0, idx1, w0b, w1b)

    usage = usage_sum[0] / T
    ideal = 1.0 / NUM_EXPERTS
    lb_loss = LB_COEF * jnp.mean((usage - ideal) ** 2)
    return out.reshape(B, S, D), lb_loss
